# baseline (speedup 1.0000x reference)
import jax
import jax.numpy as jnp
from jax import lax
from jax.experimental import pallas as pl
from jax.experimental.pallas import tpu as pltpu
from jax.experimental.pallas import tpu_sc as plsc

SEQ = 128
DIM = 1280
NC = 2
NS = 16
NW = NC * NS
RCH = SEQ // NW
PAD = 8
REP = 4
DEPTH = 8


def _sc_gather_body(table_hbm, idx_hbm, rows_hbm, idx_v, rows_v, sem):
    w = lax.axis_index("s") * NC + lax.axis_index("c")
    pltpu.sync_copy(idx_hbm.at[w], idx_v)
    pltpu.async_copy(table_hbm.at[idx_v], rows_v, sem).wait()
    pltpu.sync_copy(rows_v.at[pl.ds(0, RCH)], rows_hbm.at[pl.ds(w * RCH, RCH)])


def _tc_broadcast_body(rows_hbm, out_hbm, rep_v, in_sem, out_sem):
    stages = [
        pltpu.make_async_copy(rows_hbm, rep_v.at[k], in_sem) for k in range(REP)
    ]
    for s in stages:
        s.start()
    for s in stages:
        s.wait()
    n = SEQ // REP
    copies = [
        pltpu.make_async_copy(rep_v, out_hbm.at[pl.ds(i * REP, REP)], out_sem)
        for i in range(n)
    ]
    for i in range(n):
        if i >= DEPTH:
            copies[i - DEPTH].wait()
        copies[i].start()
    for i in range(n - DEPTH, n):
        copies[i].wait()


def kernel(inputs, length, table):
    del inputs
    idx = jnp.arange(SEQ, dtype=jnp.int32) + jnp.asarray(length, jnp.int32)
    idx = jnp.clip(idx, 0, SEQ - 1).reshape(NW, RCH)
    idx = jnp.concatenate([idx, idx], axis=1)

    gather = pl.kernel(
        _sc_gather_body,
        mesh=plsc.VectorSubcoreMesh(core_axis_name="c", subcore_axis_name="s"),
        out_type=jax.ShapeDtypeStruct((SEQ, DIM), jnp.float32),
        scratch_types=[
            pltpu.VMEM((PAD,), jnp.int32),
            pltpu.VMEM((PAD, DIM), jnp.float32),
            pltpu.SemaphoreType.DMA,
        ],
    )
    rows = gather(table, idx)

    return pl.pallas_call(
        _tc_broadcast_body,
        in_specs=[pl.BlockSpec(memory_space=pltpu.MemorySpace.HBM)],
        out_specs=pl.BlockSpec(memory_space=pltpu.MemorySpace.HBM),
        out_shape=jax.ShapeDtypeStruct((SEQ, SEQ, DIM), jnp.float32),
        scratch_shapes=[
            pltpu.VMEM((REP, SEQ, DIM), jnp.float32),
            pltpu.SemaphoreType.DMA,
            pltpu.SemaphoreType.DMA,
        ],
    )(rows)


# speedup vs baseline: 1.0797x; 1.0797x over previous
import jax
import jax.numpy as jnp
from jax import lax
from jax.experimental import pallas as pl
from jax.experimental.pallas import tpu as pltpu
from jax.experimental.pallas import tpu_sc as plsc

SEQ = 128
DIM = 1280
NS = 16
RCH = SEQ // NS
IBLK = 8


def _sc_gather_body(table_hbm, idx_hbm, rows_hbm, idx_v, rows_v, sem):
    w = lax.axis_index("s")
    pltpu.sync_copy(idx_hbm.at[w], idx_v)
    pltpu.async_copy(table_hbm.at[idx_v], rows_v, sem).wait()
    pltpu.sync_copy(rows_v, rows_hbm.at[pl.ds(w * RCH, RCH)])


def _tc_broadcast_body(rows_ref, out_ref):
    out_ref[...] = jnp.broadcast_to(rows_ref[...], (IBLK, SEQ, DIM))


def kernel(inputs, length, table):
    del inputs
    idx = jnp.arange(SEQ, dtype=jnp.int32) + jnp.asarray(length, jnp.int32)
    idx = jnp.clip(idx, 0, SEQ - 1).reshape(NS, RCH)

    gather = pl.kernel(
        _sc_gather_body,
        mesh=plsc.VectorSubcoreMesh(
            core_axis_name="c", subcore_axis_name="s", num_cores=1
        ),
        out_type=jax.ShapeDtypeStruct((SEQ, DIM), jnp.float32),
        scratch_types=[
            pltpu.VMEM((RCH,), jnp.int32),
            pltpu.VMEM((RCH, DIM), jnp.float32),
            pltpu.SemaphoreType.DMA,
        ],
    )
    rows = gather(table, idx)

    return pl.pallas_call(
        _tc_broadcast_body,
        grid=(SEQ // IBLK,),
        in_specs=[pl.BlockSpec((SEQ, DIM), lambda i: (0, 0))],
        out_specs=pl.BlockSpec((IBLK, SEQ, DIM), lambda i: (i, 0, 0)),
        out_shape=jax.ShapeDtypeStruct((SEQ, SEQ, DIM), jnp.float32),
    )(rows)
